# baseline (speedup 1.0000x reference)
import functools

import jax
import jax.numpy as jnp
from jax.experimental import pallas as pl
from jax.experimental.pallas import tpu as pltpu

B_N = 9


def _body(batch, hw, nsteps, tile, fv_ref, mb_ref, pix_ref, img_ref,
          acc_ref, bank_ref):
    i = pl.program_id(0)
    fv = fv_ref[...]
    q, c = fv.shape

    @pl.when(i < nsteps)
    def _():
        mb = mb_ref[...]
        bank_ref[pl.ds(i * tile, tile), :] = mb
        prod2 = jax.lax.dot_general(fv * -2.0, mb,
                                    (((1,), (1,)), ((), ())))
        mbn = jax.lax.dot_general(jnp.ones((1, c), fv.dtype), mb * mb,
                                  (((1,), (1,)), ((), ())))
        part = prod2[:, 0:c] + mbn[:, 0:c]
        for k in range(1, tile // c):
            part = jnp.minimum(part, prod2[:, k * c:(k + 1) * c]
                               + mbn[:, k * c:(k + 1) * c])
        prev = jnp.where(i == 0, jnp.inf, acc_ref[...])
        acc_ref[...] = jnp.minimum(prev, part)

    @pl.when(i == nsteps)
    def _():
        big = jnp.int32(2 ** 30)
        fvn = jnp.sum(fv * fv, axis=1, keepdims=True)
        mnd = jnp.min(acc_ref[...], axis=1, keepdims=True) + fvn
        s = jnp.sqrt(jnp.maximum(mnd, 0.0))
        pix_ref[...] = s

        row_iota = jax.lax.broadcasted_iota(jnp.int32, (q, 1), 0)
        col_b = jax.lax.broadcasted_iota(jnp.int32, (q, batch), 1)
        in_b = (row_iota >= col_b * hw) & (row_iota < (col_b + 1) * hw)
        sb = jnp.where(in_b, s, -jnp.inf)
        mx = jnp.max(sb, axis=0, keepdims=True)
        idx = jnp.min(jnp.where(sb == mx, row_iota, big),
                      axis=0, keepdims=True)
        onehot = (row_iota == idx).astype(fv.dtype)
        sel = jax.lax.dot_general(onehot, fv, (((0,), (0,)), ((), ())))

        bank = bank_ref[...]
        mbn = jax.lax.dot_general(jnp.ones((1, c), fv.dtype), bank * bank,
                                  (((1,), (1,)), ((), ())))
        prod2 = jax.lax.dot_general(sel * -2.0, bank,
                                    (((1,), (1,)), ((), ())))
        seln = jnp.sum(sel * sel, axis=1, keepdims=True)
        d = jnp.maximum(seln + mbn + prod2, 0.0)

        col_iota = jax.lax.broadcasted_iota(jnp.int32, d.shape, 1)
        sds = []
        for _ in range(B_N):
            mn = jnp.min(d, axis=1, keepdims=True)
            sds.append(jnp.sqrt(mn))
            amn = jnp.min(jnp.where(d == mn, col_iota, big),
                          axis=1, keepdims=True)
            d = jnp.where(col_iota == amn, jnp.inf, d)

        top = sds[-1]
        esum = jnp.zeros_like(top)
        for sd in sds:
            esum = esum + jnp.exp(sd - top)
        p0 = jnp.exp(sds[0] - top) / esum
        img_ref[...] = sds[0] * (1.0 - p0)


def kernel(feature_batch, mb):
    batch, height, width, channels = feature_batch.shape
    hw = height * width
    q = batch * hw
    m = mb.shape[0]
    fv = jnp.reshape(feature_batch, (q, channels))

    tile = 2048
    nsteps = m // tile
    pix, img = pl.pallas_call(
        functools.partial(_body, batch, hw, nsteps, tile),
        grid=(nsteps + 1,),
        in_specs=[
            pl.BlockSpec((q, channels), lambda i: (0, 0)),
            pl.BlockSpec((tile, channels),
                         lambda i: (jnp.minimum(i, nsteps - 1), 0)),
        ],
        out_specs=[
            pl.BlockSpec((q, 1), lambda i: (0, 0)),
            pl.BlockSpec((batch, 1), lambda i: (0, 0)),
        ],
        out_shape=[
            jax.ShapeDtypeStruct((q, 1), fv.dtype),
            jax.ShapeDtypeStruct((batch, 1), fv.dtype),
        ],
        scratch_shapes=[
            pltpu.VMEM((q, channels), fv.dtype),
            pltpu.VMEM((m, channels), fv.dtype),
        ],
    )(fv, mb)

    pixel_scores = jnp.reshape(pix, (batch, 1, height, width))
    image_scores = jnp.reshape(img, (batch,))
    return (pixel_scores, image_scores)


# speedup vs baseline: 1.6460x; 1.6460x over previous
import functools

import jax
import jax.numpy as jnp
from jax.experimental import pallas as pl
from jax.experimental.pallas import tpu as pltpu

B_N = 9


def _body(batch, hw, nsteps, tile, fv_ref, mb_ref, pix_ref, img_ref,
          acc_ref, bank_ref):
    i = pl.program_id(0)
    fv = fv_ref[...]
    q, c = fv.shape

    @pl.when(i < nsteps)
    def _():
        mb = mb_ref[...]
        bank_ref[pl.ds(i * tile, tile), :] = mb
        prod2 = jax.lax.dot_general(fv * -2.0, mb,
                                    (((1,), (1,)), ((), ())))
        mbn = jax.lax.dot_general(jnp.ones((1, c), fv.dtype), mb * mb,
                                  (((1,), (1,)), ((), ())))
        tt = prod2 + mbn
        parts = [tt[:, k * c:(k + 1) * c] for k in range(tile // c)]
        while len(parts) > 1:
            parts = [jnp.minimum(parts[j], parts[j + 1])
                     for j in range(0, len(parts) - 1, 2)] + (
                         [parts[-1]] if len(parts) % 2 else [])
        part = parts[0]
        prev = jnp.where(i == 0, jnp.inf, acc_ref[...])
        acc_ref[...] = jnp.minimum(prev, part)

    @pl.when(i == nsteps)
    def _():
        big = jnp.int32(2 ** 30)
        fvn = jnp.sum(fv * fv, axis=1, keepdims=True)
        mnd = jnp.min(acc_ref[...], axis=1, keepdims=True) + fvn
        s = jnp.sqrt(jnp.maximum(mnd, 0.0))
        pix_ref[...] = s

        row_iota = jax.lax.broadcasted_iota(jnp.int32, (q, 1), 0)
        col_b = jax.lax.broadcasted_iota(jnp.int32, (q, batch), 1)
        in_b = (row_iota >= col_b * hw) & (row_iota < (col_b + 1) * hw)
        sb = jnp.where(in_b, s, -jnp.inf)
        mx = jnp.max(sb, axis=0, keepdims=True)
        idx = jnp.min(jnp.where(sb == mx, row_iota, big),
                      axis=0, keepdims=True)
        onehot = (row_iota == idx).astype(fv.dtype)
        sel = jax.lax.dot_general(onehot, fv, (((0,), (0,)), ((), ())))

        bank = bank_ref[...]
        mbn = jax.lax.dot_general(jnp.ones((1, c), fv.dtype), bank * bank,
                                  (((1,), (1,)), ((), ())))
        prod2 = jax.lax.dot_general(sel * -2.0, bank,
                                    (((1,), (1,)), ((), ())))
        seln = jnp.sum(sel * sel, axis=1, keepdims=True)
        d = jnp.maximum(seln + mbn + prod2, 0.0)

        col_iota = jax.lax.broadcasted_iota(jnp.int32, d.shape, 1)
        sds = []
        for _ in range(B_N):
            mn = jnp.min(d, axis=1, keepdims=True)
            sds.append(jnp.sqrt(mn))
            amn = jnp.min(jnp.where(d == mn, col_iota, big),
                          axis=1, keepdims=True)
            d = jnp.where(col_iota == amn, jnp.inf, d)

        top = sds[-1]
        esum = jnp.zeros_like(top)
        for sd in sds:
            esum = esum + jnp.exp(sd - top)
        p0 = jnp.exp(sds[0] - top) / esum
        img_ref[...] = sds[0] * (1.0 - p0)


def kernel(feature_batch, mb):
    batch, height, width, channels = feature_batch.shape
    hw = height * width
    q = batch * hw
    m = mb.shape[0]
    fv = jnp.reshape(feature_batch, (q, channels))

    tile = 2048
    nsteps = m // tile
    pix, img = pl.pallas_call(
        functools.partial(_body, batch, hw, nsteps, tile),
        grid=(nsteps + 1,),
        in_specs=[
            pl.BlockSpec((q, channels), lambda i: (0, 0)),
            pl.BlockSpec((tile, channels),
                         lambda i: (jnp.minimum(i, nsteps - 1), 0)),
        ],
        out_specs=[
            pl.BlockSpec((q, 1), lambda i: (0, 0)),
            pl.BlockSpec((batch, 1), lambda i: (0, 0)),
        ],
        out_shape=[
            jax.ShapeDtypeStruct((q, 1), fv.dtype),
            jax.ShapeDtypeStruct((batch, 1), fv.dtype),
        ],
        scratch_shapes=[
            pltpu.VMEM((q, channels), fv.dtype),
            pltpu.VMEM((m, channels), fv.dtype),
        ],
    )(fv, mb)

    pixel_scores = jnp.reshape(pix, (batch, 1, height, width))
    image_scores = jnp.reshape(img, (batch,))
    return (pixel_scores, image_scores)


# speedup vs baseline: 1.6834x; 1.0227x over previous
import functools

import jax
import jax.numpy as jnp
from jax.experimental import pallas as pl
from jax.experimental.pallas import tpu as pltpu

B_N = 9


def _body(batch, hw, nsteps, tile, fv_ref, mb_ref, pix_ref, img_ref,
          acc_ref, bank_ref, fv2_ref):
    i = pl.program_id(0)
    fv = fv_ref[...]
    q, c = fv.shape

    @pl.when(i == 0)
    def _():
        fv2_ref[...] = fv * -2.0

    @pl.when(i < nsteps)
    def _():
        mb = mb_ref[...]
        bank_ref[pl.ds(i * tile, tile), :] = mb
        prod2 = jax.lax.dot_general(fv2_ref[...], mb,
                                    (((1,), (1,)), ((), ())))
        mbn = jax.lax.dot_general(jnp.ones((1, c), fv.dtype), mb * mb,
                                  (((1,), (1,)), ((), ())))
        tt = prod2 + mbn
        parts = [tt[:, k * c:(k + 1) * c] for k in range(tile // c)]
        while len(parts) > 1:
            parts = [jnp.minimum(parts[j], parts[j + 1])
                     for j in range(0, len(parts) - 1, 2)] + (
                         [parts[-1]] if len(parts) % 2 else [])
        part = parts[0]
        prev = jnp.where(i == 0, jnp.inf, acc_ref[...])
        acc_ref[...] = jnp.minimum(prev, part)

    @pl.when(i == nsteps)
    def _():
        big = jnp.int32(2 ** 30)
        fvn = jnp.sum(fv * fv, axis=1, keepdims=True)
        mnd = jnp.min(acc_ref[...], axis=1, keepdims=True) + fvn
        s = jnp.sqrt(jnp.maximum(mnd, 0.0))
        pix_ref[...] = s

        row_iota = jax.lax.broadcasted_iota(jnp.int32, (q, 1), 0)
        col_b = jax.lax.broadcasted_iota(jnp.int32, (q, batch), 1)
        in_b = (row_iota >= col_b * hw) & (row_iota < (col_b + 1) * hw)
        sb = jnp.where(in_b, s, -jnp.inf)
        mx = jnp.max(sb, axis=0, keepdims=True)
        idx = jnp.min(jnp.where(sb == mx, row_iota, big),
                      axis=0, keepdims=True)
        onehot = (row_iota == idx).astype(fv.dtype)
        sel = jax.lax.dot_general(onehot, fv, (((0,), (0,)), ((), ())))

        bank = bank_ref[...]
        mbn = jax.lax.dot_general(jnp.ones((1, c), fv.dtype), bank * bank,
                                  (((1,), (1,)), ((), ())))
        prod2 = jax.lax.dot_general(sel * -2.0, bank,
                                    (((1,), (1,)), ((), ())))
        seln = jnp.sum(sel * sel, axis=1, keepdims=True)
        d = jnp.maximum(seln + mbn + prod2, 0.0)

        col_iota = jax.lax.broadcasted_iota(jnp.int32, d.shape, 1)
        sds = []
        for _ in range(B_N):
            mn = jnp.min(d, axis=1, keepdims=True)
            sds.append(jnp.sqrt(mn))
            amn = jnp.min(jnp.where(d == mn, col_iota, big),
                          axis=1, keepdims=True)
            d = jnp.where(col_iota == amn, jnp.inf, d)

        top = sds[-1]
        esum = jnp.zeros_like(top)
        for sd in sds:
            esum = esum + jnp.exp(sd - top)
        p0 = jnp.exp(sds[0] - top) / esum
        img_ref[...] = sds[0] * (1.0 - p0)


def kernel(feature_batch, mb):
    batch, height, width, channels = feature_batch.shape
    hw = height * width
    q = batch * hw
    m = mb.shape[0]
    fv = jnp.reshape(feature_batch, (q, channels))

    tile = 2048
    nsteps = m // tile
    pix, img = pl.pallas_call(
        functools.partial(_body, batch, hw, nsteps, tile),
        grid=(nsteps + 1,),
        in_specs=[
            pl.BlockSpec((q, channels), lambda i: (0, 0)),
            pl.BlockSpec((tile, channels),
                         lambda i: (jnp.minimum(i, nsteps - 1), 0)),
        ],
        out_specs=[
            pl.BlockSpec((q, 1), lambda i: (0, 0)),
            pl.BlockSpec((batch, 1), lambda i: (0, 0)),
        ],
        out_shape=[
            jax.ShapeDtypeStruct((q, 1), fv.dtype),
            jax.ShapeDtypeStruct((batch, 1), fv.dtype),
        ],
        scratch_shapes=[
            pltpu.VMEM((q, channels), fv.dtype),
            pltpu.VMEM((m, channels), fv.dtype),
            pltpu.VMEM((q, channels), fv.dtype),
        ],
    )(fv, mb)

    pixel_scores = jnp.reshape(pix, (batch, 1, height, width))
    image_scores = jnp.reshape(img, (batch,))
    return (pixel_scores, image_scores)
